# baseline (speedup 1.0000x reference)
import functools

import jax
import jax.numpy as jnp
from jax.experimental import pallas as pl
from jax.experimental.pallas import tpu as pltpu


def _make_outer(n_in_blocks, in_block, k, d_in, d_out, in_bufs):
    out_block = in_block // k

    def outer(x_hbm, w_ref, b_ref, o_hbm):
        def body(idxs, x_blk, o_blk):
            _, j = idxs
            rows = x_blk[pl.ds(j * out_block, out_block), :]
            h = jnp.dot(rows, w_ref[...], preferred_element_type=jnp.float32)
            o_blk[...] = jnp.maximum(h + b_ref[...], 0.0)

        pipe = pltpu.emit_pipeline(
            body,
            grid=(n_in_blocks, k),
            in_specs=[
                pl.BlockSpec(
                    (in_block, d_in), lambda i, j: (i, 0),
                    pipeline_mode=pl.Buffered(buffer_count=in_bufs,
                                              use_lookahead=True)),
            ],
            out_specs=[
                pl.BlockSpec((out_block, d_out), lambda i, j: (i * k + j, 0)),
            ],
            _explicit_indices=True,
        )
        pipe(x_hbm, o_hbm)

    return outer


@functools.partial(jax.jit, static_argnames=())
def kernel(x, weights_encode, bias_encode):
    n, d_in = x.shape
    d_out = weights_encode.shape[1]
    in_block = 20000
    k = 4
    n_in_blocks = n // in_block
    bias2d = bias_encode.reshape(1, d_out)
    return pl.pallas_call(
        _make_outer(n_in_blocks, in_block, k, d_in, d_out, in_bufs=3),
        in_specs=[
            pl.BlockSpec(memory_space=pl.ANY),
            pl.BlockSpec(memory_space=pltpu.VMEM),
            pl.BlockSpec(memory_space=pltpu.VMEM),
        ],
        out_specs=pl.BlockSpec(memory_space=pl.ANY),
        out_shape=jax.ShapeDtypeStruct((n, d_out), jnp.float32),
    )(x, weights_encode, bias2d)


# speedup vs baseline: 1.0314x; 1.0314x over previous
import functools

import jax
import jax.numpy as jnp
from jax.experimental import pallas as pl
from jax.experimental.pallas import tpu as pltpu


def _make_outer(num_blocks, block_n, d_in, d_out, in_bufs):
    def outer(x_hbm, w_ref, b_ref, o_hbm):
        def body(x_blk, o_blk):
            h = jnp.dot(x_blk[...].astype(jnp.bfloat16), w_ref[...],
                        preferred_element_type=jnp.float32)
            o_blk[...] = jnp.maximum(h + b_ref[...], 0.0)

        pipe = pltpu.emit_pipeline(
            body,
            grid=(num_blocks,),
            in_specs=[
                pl.BlockSpec((block_n, d_in), lambda i: (i, 0),
                             pipeline_mode=pl.Buffered(buffer_count=in_bufs)),
            ],
            out_specs=[
                pl.BlockSpec((block_n, d_out), lambda i: (i, 0)),
            ],
        )
        pipe(x_hbm, o_hbm)

    return outer


@functools.partial(jax.jit, static_argnames=())
def kernel(x, weights_encode, bias_encode):
    n, d_in = x.shape
    d_out = weights_encode.shape[1]
    block_n = 16000
    num_blocks = pl.cdiv(n, block_n)
    bias2d = bias_encode.reshape(1, d_out)
    w_bf16 = weights_encode.astype(jnp.bfloat16)
    return pl.pallas_call(
        _make_outer(num_blocks, block_n, d_in, d_out, in_bufs=4),
        in_specs=[
            pl.BlockSpec(memory_space=pl.ANY),
            pl.BlockSpec(memory_space=pltpu.VMEM),
            pl.BlockSpec(memory_space=pltpu.VMEM),
        ],
        out_specs=pl.BlockSpec(memory_space=pl.ANY),
        out_shape=jax.ShapeDtypeStruct((n, d_out), jnp.float32),
    )(x, w_bf16, bias2d)


# speedup vs baseline: 1.0791x; 1.0462x over previous
import functools

import jax
import jax.numpy as jnp
from jax.experimental import pallas as pl
from jax.experimental.pallas import tpu as pltpu


def _make_outer(num_blocks, block_n, d_in, d_out, in_bufs):
    def outer(x_hbm, w_ref, b_ref, o_hbm):
        def body(x_blk, o_blk):
            h = jnp.dot(x_blk[...], w_ref[...],
                        preferred_element_type=jnp.float32)
            o_blk[...] = jnp.maximum(h + b_ref[...], 0.0)

        pipe = pltpu.emit_pipeline(
            body,
            grid=(num_blocks,),
            in_specs=[
                pl.BlockSpec((block_n, d_in), lambda i: (i, 0),
                             pipeline_mode=pl.Buffered(buffer_count=in_bufs)),
            ],
            out_specs=[
                pl.BlockSpec((block_n, d_out), lambda i: (i, 0)),
            ],
        )
        pipe(x_hbm, o_hbm)

    return outer


@functools.partial(jax.jit, static_argnames=())
def kernel(x, weights_encode, bias_encode):
    n, d_in = x.shape
    d_out = weights_encode.shape[1]
    block_n = 16000
    num_blocks = pl.cdiv(n, block_n)
    bias2d = bias_encode.reshape(1, d_out)
    return pl.pallas_call(
        _make_outer(num_blocks, block_n, d_in, d_out, in_bufs=4),
        in_specs=[
            pl.BlockSpec(memory_space=pl.ANY),
            pl.BlockSpec(memory_space=pltpu.VMEM),
            pl.BlockSpec(memory_space=pltpu.VMEM),
        ],
        out_specs=pl.BlockSpec(memory_space=pl.ANY),
        out_shape=jax.ShapeDtypeStruct((n, d_out), jnp.float32),
    )(x, weights_encode, bias2d)
